# baseline (speedup 1.0000x reference)
import functools

import numpy as np
import jax
import jax.numpy as jnp
from jax import lax
from jax.experimental import pallas as pl
from jax.experimental.pallas import tpu as pltpu
from jax.experimental.pallas import tpu_sc as plsc

N = 10000
E = 160000
D = 256
L = 3

NT = 32
NPT = 313
NROWS = NT * NPT
JUNK = NPT
CAP = 5632
ECH = 1600
CH = 128
NCH = CAP // CH

ROW_BLK = 1000

_mesh = plsc.VectorSubcoreMesh(core_axis_name="c", subcore_axis_name="s")
_sc_params = pltpu.CompilerParams(needs_layout_passes=False)


def _wid():
    return lax.axis_index("s") * 2 + lax.axis_index("c")


def _const_table():
    rows = [np.full(16, 15)]
    for k in range(4):
        rows.append(np.maximum(np.arange(16) - (1 << k), 0))
    for k in range(4):
        rows.append((np.arange(16) >= (1 << k)).astype(np.int64))
    for w in range(NT):
        rows.append(np.full(16, w * NPT))
    return jnp.asarray(np.concatenate(rows).astype(np.int32))


TAB_ROWS = 9 + NT


@functools.partial(
    pl.kernel,
    mesh=_mesh,
    compiler_params=_sc_params,
    out_type=[
        jax.ShapeDtypeStruct((NT * CAP,), jnp.int32),
        jax.ShapeDtypeStruct((NT * CAP,), jnp.int32),
    ],
    scratch_types=[
        pltpu.VMEM((ECH,), jnp.int32),
        pltpu.VMEM((ECH,), jnp.int32),
        pltpu.VMEM((CAP,), jnp.int32),
        pltpu.VMEM((CAP,), jnp.int32),
        pltpu.VMEM((TAB_ROWS * 16,), jnp.int32),
    ],
)
def _edge_partition(src_hbm, dst_hbm, tab_hbm, srcs_out, ldst_out,
                    sv, dv, osrc, oldst, tv):
    wid = _wid()
    pltpu.sync_copy(tab_hbm, tv)
    splat15 = tv[pl.ds(0, 16)]
    shs = [tv[pl.ds((1 + k) * 16, 16)] for k in range(4)]
    zms = [tv[pl.ds((5 + k) * 16, 16)] for k in range(4)]
    lo_vec = tv[pl.ds((9 + wid) * 16, 16)]

    zero16 = lax.iota(jnp.int32, 16) * 0
    junk16 = zero16 + JUNK

    def init_body(i, _):
        osrc[pl.ds(i * 16, 16)] = zero16
        oldst[pl.ds(i * 16, 16)] = junk16
        return 0

    lax.fori_loop(0, CAP // 16, init_body, 0)

    def chunk_body(k, cnt_vec):
        pltpu.sync_copy(src_hbm.at[pl.ds(k * ECH, ECH)], sv)
        pltpu.sync_copy(dst_hbm.at[pl.ds(k * ECH, ECH)], dv)

        def grp_body(q, cnt_vec):
            d = dv[pl.ds(q * 16, 16)]
            s = sv[pl.ds(q * 16, 16)]
            ld = d - lo_vec
            m = (ld >= 0) & (ld < NPT)
            ps = m.astype(jnp.int32)
            for k4 in range(4):
                ps = ps + jnp.take(ps, shs[k4]) * zms[k4]
            offs = cnt_vec + ps - 1
            plsc.store_scatter(osrc, [offs], s, mask=m)
            plsc.store_scatter(oldst, [offs], ld, mask=m)
            return cnt_vec + jnp.take(ps, splat15)

        return lax.fori_loop(0, ECH // 16, grp_body, cnt_vec)

    lax.fori_loop(0, E // ECH, chunk_body, zero16)
    pltpu.sync_copy(osrc, srcs_out.at[pl.ds(wid * CAP, CAP)])
    pltpu.sync_copy(oldst, ldst_out.at[pl.ds(wid * CAP, CAP)])


@functools.partial(
    pl.kernel,
    mesh=_mesh,
    compiler_params=_sc_params,
    out_type=jax.ShapeDtypeStruct((NROWS * D,), jnp.float32),
    scratch_types=[
        pltpu.VMEM(((NPT + 1) * D,), jnp.float32),
        pltpu.VMEM((CH, D), jnp.float32),
        pltpu.VMEM((CAP,), jnp.int32),
        pltpu.VMEM((CAP,), jnp.int32),
        pltpu.SemaphoreType.DMA,
    ],
)
def _segsum(h_hbm, srcs_hbm, ldst_hbm, agg_out, acc, stg, sidx, ldv, sem):
    wid = _wid()
    zero16 = lax.iota(jnp.int32, 16) * 0
    fzero16 = zero16.astype(jnp.float32)

    def zero_body(i, _):
        acc[pl.ds(i * 16, 16)] = fzero16
        return 0

    lax.fori_loop(0, (NPT + 1) * D // 16, zero_body, 0)

    pltpu.sync_copy(srcs_hbm.at[pl.ds(wid * CAP, CAP)], sidx)
    pltpu.sync_copy(ldst_hbm.at[pl.ds(wid * CAP, CAP)], ldv)

    evs = [lax.iota(jnp.int32, 16) + q * 16 for q in range(CH // 16)]

    def chunk_body(g, _):
        pltpu.async_copy(h_hbm.at[sidx.at[pl.ds(g * CH, CH)]], stg, sem).wait()
        dvs = [ldv[pl.ds(g * CH + q * 16, 16)] * D for q in range(CH // 16)]

        def col_body(col, colv):
            for q in range(CH // 16):
                vals = plsc.load_gather(stg, [evs[q], colv])
                plsc.addupdate_scatter(acc, [dvs[q] + colv], vals)
            return colv + 1

        lax.fori_loop(0, D, col_body, zero16)
        return 0

    lax.fori_loop(0, NCH, chunk_body, 0)
    pltpu.sync_copy(acc.at[pl.ds(0, NPT * D)],
                    agg_out.at[pl.ds(wid * NPT * D, NPT * D)])


B_TOT = 16384
B_PER_W = B_TOT // NT


@functools.partial(
    pl.kernel,
    mesh=_mesh,
    compiler_params=_sc_params,
    out_type=jax.ShapeDtypeStruct((B_TOT, D), jnp.float32),
    scratch_types=[
        pltpu.VMEM((B_PER_W,), jnp.int32),
        pltpu.VMEM((CH, D), jnp.float32),
        pltpu.SemaphoreType.DMA,
    ],
)
def _take_rows(node_hbm, flat_hbm, out_hbm, idx_v, stg, sem):
    wid = _wid()
    base = wid * B_PER_W
    pltpu.sync_copy(flat_hbm.at[pl.ds(base, B_PER_W)], idx_v)
    for cc in range(B_PER_W // CH):
        pltpu.async_copy(
            node_hbm.at[idx_v.at[pl.ds(cc * CH, CH)]], stg, sem).wait()
        pltpu.sync_copy(stg, out_hbm.at[pl.ds(base + cc * CH, CH)])


def _gin_layer_body(scale_ref, h_ref, agg_ref, w_ref, b_ref, r_ref, rb_ref, out_ref):
    x = scale_ref[0, 0] * h_ref[...] + agg_ref[...]
    h1 = jnp.dot(x, w_ref[...], preferred_element_type=jnp.float32) + b_ref[...]
    hr = jnp.maximum(h1, 0.0)
    h2 = jnp.dot(hr, r_ref[...], preferred_element_type=jnp.float32) + rb_ref[...]
    out_ref[...] = hr + jnp.maximum(h2, 0.0)


def _gin_layer3_attn_body(scale_ref, h_ref, agg_ref, w_ref, b_ref, r_ref, rb_ref,
                          h1_ref, h2_ref, aw_ref, ab_ref, out_ref):
    x = scale_ref[0, 0] * h_ref[...] + agg_ref[...]
    t1 = jnp.dot(x, w_ref[...], preferred_element_type=jnp.float32) + b_ref[...]
    hr = jnp.maximum(t1, 0.0)
    t2 = jnp.dot(hr, r_ref[...], preferred_element_type=jnp.float32) + rb_ref[...]
    h3 = hr + jnp.maximum(t2, 0.0)
    h1 = h1_ref[...]
    h2 = h2_ref[...]
    aw = aw_ref[...]
    ab = ab_ref[0, 0]
    s1 = jnp.sum(h1 * aw, axis=1, keepdims=True) + ab
    s2 = jnp.sum(h2 * aw, axis=1, keepdims=True) + ab
    s3 = jnp.sum(h3 * aw, axis=1, keepdims=True) + ab
    m = jnp.maximum(jnp.maximum(s1, s2), s3)
    e1 = jnp.exp(s1 - m)
    e2 = jnp.exp(s2 - m)
    e3 = jnp.exp(s3 - m)
    denom = e1 + e2 + e3
    out_ref[...] = (e1 * h1 + e2 * h2 + e3 * h3) / denom


def _row_spec():
    return pl.BlockSpec((ROW_BLK, D), lambda i: (i, 0))


def _full_spec(shape):
    return pl.BlockSpec(shape, lambda i: tuple(0 for _ in shape))


def _smem_spec(shape):
    return pl.BlockSpec(shape, lambda i: tuple(0 for _ in shape),
                        memory_space=pltpu.SMEM)


def _gin_layer(scale, h, agg, w, b, r, rb):
    grid = (N // ROW_BLK,)
    return pl.pallas_call(
        _gin_layer_body,
        grid=grid,
        in_specs=[
            _smem_spec((1, 1)),
            _row_spec(), _row_spec(),
            _full_spec((D, D)), _full_spec((1, D)),
            _full_spec((D, D)), _full_spec((1, D)),
        ],
        out_specs=_row_spec(),
        out_shape=jax.ShapeDtypeStruct((N, D), jnp.float32),
    )(scale, h, agg, w, b.reshape(1, D), r, rb.reshape(1, D))


def _gin_layer3_attn(scale, h, agg, w, b, r, rb, h1, h2, aw, ab):
    grid = (N // ROW_BLK,)
    return pl.pallas_call(
        _gin_layer3_attn_body,
        grid=grid,
        in_specs=[
            _smem_spec((1, 1)),
            _row_spec(), _row_spec(),
            _full_spec((D, D)), _full_spec((1, D)),
            _full_spec((D, D)), _full_spec((1, D)),
            _row_spec(), _row_spec(),
            _full_spec((1, D)), _smem_spec((1, 1)),
        ],
        out_specs=_row_spec(),
        out_shape=jax.ShapeDtypeStruct((N, D), jnp.float32),
    )(scale, h, agg, w, b.reshape(1, D), r, rb.reshape(1, D),
      h1, h2, aw.reshape(1, D), ab.reshape(1, 1))


def kernel(inputs, edge_index, emb, eps,
           W0, b0, W1, b1, W2, b2,
           R0, rb0, R1, rb1, R2, rb2,
           attn_W, attn_b):
    src = edge_index[0]
    dst = edge_index[1]
    Ws = [(W0, b0), (W1, b1), (W2, b2)]
    Rs = [(R0, rb0), (R1, rb1), (R2, rb2)]

    tab = _const_table()
    srcs, ldst = _edge_partition(src, dst, tab)

    h = emb
    hidden = []
    node_out = None
    for i in range(L):
        agg = _segsum(h, srcs, ldst).reshape(NROWS, D)
        scale = (1.0 + eps[i]).reshape(1, 1)
        if i < L - 1:
            h = _gin_layer(scale, h, agg, Ws[i][0], Ws[i][1], Rs[i][0], Rs[i][1])
            hidden.append(h)
        else:
            node_out = _gin_layer3_attn(scale, h, agg, Ws[i][0], Ws[i][1],
                                        Rs[i][0], Rs[i][1],
                                        hidden[0], hidden[1], attn_W, attn_b)

    flat = inputs.reshape(-1)
    out = _take_rows(node_out, flat)
    return out.reshape(inputs.shape + (D,))


# speedup vs baseline: 2.1231x; 2.1231x over previous
import functools

import numpy as np
import jax
import jax.numpy as jnp
from jax import lax
from jax.experimental import pallas as pl
from jax.experimental.pallas import tpu as pltpu
from jax.experimental.pallas import tpu_sc as plsc

N = 10000
E = 160000
D = 256
L = 3

NT = 32
NPT = 313
NROWS = NT * NPT
JUNK = NPT
CAP = 5632
ECH = 8000
CH = 64
NCH = CAP // CH

ROW_BLK = 1000

_mesh = plsc.VectorSubcoreMesh(core_axis_name="c", subcore_axis_name="s")
_sc_params = pltpu.CompilerParams(needs_layout_passes=False)


def _wid():
    return lax.axis_index("s") * 2 + lax.axis_index("c")


def _const_table():
    rows = [np.full(16, 15)]
    for k in range(4):
        rows.append(np.maximum(np.arange(16) - (1 << k), 0))
    for k in range(4):
        rows.append((np.arange(16) >= (1 << k)).astype(np.int64))
    for w in range(NT):
        rows.append(np.full(16, w * NPT))
    return jnp.asarray(np.concatenate(rows).astype(np.int32))


TAB_ROWS = 9 + NT


@functools.partial(
    pl.kernel,
    mesh=_mesh,
    compiler_params=_sc_params,
    out_type=[
        jax.ShapeDtypeStruct((NT * CAP,), jnp.int32),
        jax.ShapeDtypeStruct((NT * CAP,), jnp.int32),
    ],
    scratch_types=[
        pltpu.VMEM((ECH,), jnp.int32),
        pltpu.VMEM((ECH,), jnp.int32),
        pltpu.VMEM((CAP,), jnp.int32),
        pltpu.VMEM((CAP,), jnp.int32),
        pltpu.VMEM((TAB_ROWS * 16,), jnp.int32),
    ],
)
def _edge_partition(src_hbm, dst_hbm, tab_hbm, srcs_out, ldst_out,
                    sv, dv, osrc, oldst, tv):
    wid = _wid()
    pltpu.sync_copy(tab_hbm, tv)
    splat15 = tv[pl.ds(0, 16)]
    shs = [tv[pl.ds((1 + k) * 16, 16)] for k in range(4)]
    zms = [tv[pl.ds((5 + k) * 16, 16)] for k in range(4)]
    lo_vec = tv[pl.ds((9 + wid) * 16, 16)]

    zero16 = lax.iota(jnp.int32, 16) * 0
    junk16 = zero16 + JUNK

    def init_body(i, _):
        osrc[pl.ds(i * 16, 16)] = zero16
        oldst[pl.ds(i * 16, 16)] = junk16
        return 0

    lax.fori_loop(0, CAP // 16, init_body, 0)

    def chunk_body(k, cnt_vec):
        pltpu.sync_copy(src_hbm.at[pl.ds(k * ECH, ECH)], sv)
        pltpu.sync_copy(dst_hbm.at[pl.ds(k * ECH, ECH)], dv)

        def grp_body(q, cnt_vec):
            d = dv[pl.ds(q * 16, 16)]
            s = sv[pl.ds(q * 16, 16)]
            ld = d - lo_vec
            m = (ld >= 0) & (ld < NPT)
            ps = m.astype(jnp.int32)
            for k4 in range(4):
                ps = ps + jnp.take(ps, shs[k4]) * zms[k4]
            offs = cnt_vec + ps - 1
            plsc.store_scatter(osrc, [offs], s, mask=m)
            plsc.store_scatter(oldst, [offs], ld, mask=m)
            return cnt_vec + jnp.take(ps, splat15)

        return lax.fori_loop(0, ECH // 16, grp_body, cnt_vec)

    lax.fori_loop(0, E // ECH, chunk_body, zero16)
    pltpu.sync_copy(osrc, srcs_out.at[pl.ds(wid * CAP, CAP)])
    pltpu.sync_copy(oldst, ldst_out.at[pl.ds(wid * CAP, CAP)])


@functools.partial(
    pl.kernel,
    mesh=_mesh,
    compiler_params=_sc_params,
    out_type=jax.ShapeDtypeStruct((NROWS * D,), jnp.float32),
    scratch_types=[
        pltpu.VMEM(((NPT + 1) * D,), jnp.float32),
        pltpu.VMEM((CH, D), jnp.float32),
        pltpu.VMEM((CH, D), jnp.float32),
        pltpu.VMEM((CAP,), jnp.int32),
        pltpu.VMEM((CAP + 16,), jnp.int32),
        pltpu.SemaphoreType.DMA,
        pltpu.SemaphoreType.DMA,
    ],
)
def _segsum(h_hbm, srcs_hbm, ldst_hbm, agg_out,
            acc, stg0, stg1, sidx, ldv, sem0, sem1):
    wid = _wid()
    zero16 = lax.iota(jnp.int32, 16) * 0
    fzero16 = zero16.astype(jnp.float32)
    stgs = [stg0, stg1]
    sems = [sem0, sem1]

    def zero_body(i, _):
        acc[pl.ds(i * 16, 16)] = fzero16
        return 0

    lax.fori_loop(0, (NPT + 1) * D // 16, zero_body, 0)

    pltpu.sync_copy(srcs_hbm.at[pl.ds(wid * CAP, CAP)], sidx)
    pltpu.sync_copy(ldst_hbm.at[pl.ds(wid * CAP, CAP)],
                    ldv.at[pl.ds(0, CAP)])

    cvecs = [lax.iota(jnp.int32, 16) + 16 * c for c in range(16)]

    pltpu.async_copy(h_hbm.at[sidx.at[pl.ds(0, CH)]], stg0, sem0)
    pltpu.async_copy(h_hbm.at[sidx.at[pl.ds(CH, CH)]], stg1, sem1)

    def pair_body(gg, _):
        for b in range(2):
            g = gg * 2 + b
            pltpu.make_async_copy(
                h_hbm.at[sidx.at[pl.ds(g * CH, CH)]], stgs[b], sems[b]
            ).wait()

            def edge_body(j, ev_splat):
                dvx = ldv[pl.ds(g * CH + j, 16)]
                dbase = jnp.take(dvx, zero16) * D
                for c in range(16):
                    vals = plsc.load_gather(stgs[b], [ev_splat, cvecs[c]])
                    plsc.addupdate_scatter(acc, [dbase + cvecs[c]], vals)
                return ev_splat + 1

            lax.fori_loop(0, CH, edge_body, zero16)

            @pl.when(g + 2 < NCH)
            def _():
                pltpu.async_copy(
                    h_hbm.at[sidx.at[pl.ds((g + 2) * CH, CH)]],
                    stgs[b], sems[b])
        return 0

    lax.fori_loop(0, NCH // 2, pair_body, 0)
    pltpu.sync_copy(acc.at[pl.ds(0, NPT * D)],
                    agg_out.at[pl.ds(wid * NPT * D, NPT * D)])


B_TOT = 16384
B_PER_W = B_TOT // NT


@functools.partial(
    pl.kernel,
    mesh=_mesh,
    compiler_params=_sc_params,
    out_type=jax.ShapeDtypeStruct((B_TOT, D), jnp.float32),
    scratch_types=[
        pltpu.VMEM((B_PER_W,), jnp.int32),
        pltpu.VMEM((CH, D), jnp.float32),
        pltpu.SemaphoreType.DMA,
    ],
)
def _take_rows(node_hbm, flat_hbm, out_hbm, idx_v, stg, sem):
    wid = _wid()
    base = wid * B_PER_W
    pltpu.sync_copy(flat_hbm.at[pl.ds(base, B_PER_W)], idx_v)
    for cc in range(B_PER_W // CH):
        pltpu.async_copy(
            node_hbm.at[idx_v.at[pl.ds(cc * CH, CH)]], stg, sem).wait()
        pltpu.sync_copy(stg, out_hbm.at[pl.ds(base + cc * CH, CH)])


def _gin_layer_body(scale_ref, h_ref, agg_ref, w_ref, b_ref, r_ref, rb_ref, out_ref):
    x = scale_ref[0, 0] * h_ref[...] + agg_ref[...]
    h1 = jnp.dot(x, w_ref[...], preferred_element_type=jnp.float32) + b_ref[...]
    hr = jnp.maximum(h1, 0.0)
    h2 = jnp.dot(hr, r_ref[...], preferred_element_type=jnp.float32) + rb_ref[...]
    out_ref[...] = hr + jnp.maximum(h2, 0.0)


def _gin_layer3_attn_body(scale_ref, h_ref, agg_ref, w_ref, b_ref, r_ref, rb_ref,
                          h1_ref, h2_ref, aw_ref, ab_ref, out_ref):
    x = scale_ref[0, 0] * h_ref[...] + agg_ref[...]
    t1 = jnp.dot(x, w_ref[...], preferred_element_type=jnp.float32) + b_ref[...]
    hr = jnp.maximum(t1, 0.0)
    t2 = jnp.dot(hr, r_ref[...], preferred_element_type=jnp.float32) + rb_ref[...]
    h3 = hr + jnp.maximum(t2, 0.0)
    h1 = h1_ref[...]
    h2 = h2_ref[...]
    aw = aw_ref[...]
    ab = ab_ref[0, 0]
    s1 = jnp.sum(h1 * aw, axis=1, keepdims=True) + ab
    s2 = jnp.sum(h2 * aw, axis=1, keepdims=True) + ab
    s3 = jnp.sum(h3 * aw, axis=1, keepdims=True) + ab
    m = jnp.maximum(jnp.maximum(s1, s2), s3)
    e1 = jnp.exp(s1 - m)
    e2 = jnp.exp(s2 - m)
    e3 = jnp.exp(s3 - m)
    denom = e1 + e2 + e3
    out_ref[...] = (e1 * h1 + e2 * h2 + e3 * h3) / denom


def _row_spec():
    return pl.BlockSpec((ROW_BLK, D), lambda i: (i, 0))


def _full_spec(shape):
    return pl.BlockSpec(shape, lambda i: tuple(0 for _ in shape))


def _smem_spec(shape):
    return pl.BlockSpec(shape, lambda i: tuple(0 for _ in shape),
                        memory_space=pltpu.SMEM)


def _gin_layer(scale, h, agg, w, b, r, rb):
    grid = (N // ROW_BLK,)
    return pl.pallas_call(
        _gin_layer_body,
        grid=grid,
        in_specs=[
            _smem_spec((1, 1)),
            _row_spec(), _row_spec(),
            _full_spec((D, D)), _full_spec((1, D)),
            _full_spec((D, D)), _full_spec((1, D)),
        ],
        out_specs=_row_spec(),
        out_shape=jax.ShapeDtypeStruct((N, D), jnp.float32),
    )(scale, h, agg, w, b.reshape(1, D), r, rb.reshape(1, D))


def _gin_layer3_attn(scale, h, agg, w, b, r, rb, h1, h2, aw, ab):
    grid = (N // ROW_BLK,)
    return pl.pallas_call(
        _gin_layer3_attn_body,
        grid=grid,
        in_specs=[
            _smem_spec((1, 1)),
            _row_spec(), _row_spec(),
            _full_spec((D, D)), _full_spec((1, D)),
            _full_spec((D, D)), _full_spec((1, D)),
            _row_spec(), _row_spec(),
            _full_spec((1, D)), _smem_spec((1, 1)),
        ],
        out_specs=_row_spec(),
        out_shape=jax.ShapeDtypeStruct((N, D), jnp.float32),
    )(scale, h, agg, w, b.reshape(1, D), r, rb.reshape(1, D),
      h1, h2, aw.reshape(1, D), ab.reshape(1, 1))


def kernel(inputs, edge_index, emb, eps,
           W0, b0, W1, b1, W2, b2,
           R0, rb0, R1, rb1, R2, rb2,
           attn_W, attn_b):
    src = edge_index[0]
    dst = edge_index[1]
    Ws = [(W0, b0), (W1, b1), (W2, b2)]
    Rs = [(R0, rb0), (R1, rb1), (R2, rb2)]

    tab = _const_table()
    srcs, ldst = _edge_partition(src, dst, tab)

    h = emb
    hidden = []
    node_out = None
    for i in range(L):
        agg = _segsum(h, srcs, ldst).reshape(NROWS, D)
        scale = (1.0 + eps[i]).reshape(1, 1)
        if i < L - 1:
            h = _gin_layer(scale, h, agg, Ws[i][0], Ws[i][1], Rs[i][0], Rs[i][1])
            hidden.append(h)
        else:
            node_out = _gin_layer3_attn(scale, h, agg, Ws[i][0], Ws[i][1],
                                        Rs[i][0], Rs[i][1],
                                        hidden[0], hidden[1], attn_W, attn_b)

    flat = inputs.reshape(-1)
    out = _take_rows(node_out, flat)
    return out.reshape(inputs.shape + (D,))
